# baseline (speedup 1.0000x reference)
import jax
import jax.numpy as jnp
from jax.experimental import pallas as pl

_N = 48
_L = 64
_STEPS = _N - 1


def _leaky(x):
    return jnp.maximum(x, 0.01 * x)


def _prims_kernel(x0c_ref, x1c_ref, x0r_ref, x1r_ref,
                  enc_w_ref, enc_b_ref, m1w_ref, m2w_ref, uw_ref,
                  mw_ref, mb_ref, p1w_ref, pb1_ref, p2w_ref, pb2_ref,
                  out_ref):
    d0 = x0c_ref[:, :] - x0r_ref[:, :]
    d1 = x1c_ref[:, :] - x1r_ref[:, :]
    ew = jnp.sqrt(d0 * d0 + d1 * d1 + 1e-12)
    ewv = ew[:, :, None] * m1w_ref[2 * _L:, :][None, :, :]

    enc_w0 = enc_w_ref[0:1, :]
    enc_w1 = enc_w_ref[1:, :]
    enc_b = enc_b_ref[:, :]
    m1sd = jnp.concatenate([m1w_ref[_L:2 * _L, :], m1w_ref[0:_L, :]], axis=1)
    m2w = m2w_ref[:, :]
    uw = uw_ref[:, :]
    mw = mw_ref[:, :]
    mb = mb_ref[:, :]

    iota = jax.lax.broadcasted_iota(jnp.int32, (_N, 1), 0)

    def step(_, carry):
        h, pt, _enc = carry
        encoded = jnp.maximum(pt * enc_w0 + h @ enc_w1 + enc_b, 0.0)
        sd = encoded @ m1sd
        m1 = _leaky(sd[:, None, 0:_L] + sd[None, :, _L:] + ewv)
        z = m1.reshape(_N * _N, _L) @ m2w
        aggr = _leaky(jnp.max(z.reshape(_N, _N, _L), axis=0))
        h_new = jnp.clip(
            _leaky(jnp.concatenate([encoded, aggr], axis=1) @ uw), -1e9, 1e9)
        logits = jax.nn.sigmoid(
            jnp.concatenate([encoded, h_new], axis=1) @ mw + mb)
        mx = jnp.max(logits)
        idx = jnp.min(jnp.where(logits == mx, iota, _N))
        pt_new = jnp.where(iota == idx, 1.0, pt)
        return (h_new, pt_new, encoded)

    init = (jnp.zeros((_N, _L), jnp.float32),
            jnp.zeros((_N, 1), jnp.float32),
            jnp.zeros((_N, _L), jnp.float32))
    h, _pt, enc = jax.lax.fori_loop(0, _STEPS, step, init)

    eh = jnp.concatenate([enc, h], axis=1)
    s_part = eh @ p1w_ref[0:2 * _L, :]
    d_part = eh @ p1w_ref[2 * _L:, :]
    pe = jnp.maximum(
        s_part[:, None, :] + d_part[None, :, :] + pb1_ref[:, :][None, :, :],
        0.0)
    out_ref[:, :] = pe.reshape(_N * _N, _L) @ p2w_ref[:, :] + pb2_ref[:, :]


def kernel(X, enc_W, enc_b, M1_W, M2_W, U_W, mst_W, mst_b,
           pred_W1, pred_b1, pred_W2, pred_b2):
    args = (
        X[:, 0:1], X[:, 1:2],
        X[:, 0].reshape(1, _N), X[:, 1].reshape(1, _N),
        enc_W, enc_b.reshape(1, _L),
        M1_W, M2_W, U_W,
        mst_W, mst_b.reshape(1, 1),
        pred_W1, pred_b1.reshape(1, _L),
        pred_W2, pred_b2.reshape(1, 1),
    )
    out = pl.pallas_call(
        _prims_kernel,
        out_shape=jax.ShapeDtypeStruct((_N * _N, 1), jnp.float32),
    )(*args)
    return out.reshape(_N, _N)
